# baseline (speedup 1.0000x reference)
import functools

import jax
import jax.numpy as jnp
from jax import lax
from jax.experimental import pallas as pl
from jax.experimental.pallas import tpu as pltpu
from jax.experimental.pallas import tpu_sc as plsc

N = 10000
D = 256
NC = 2
NS = 16
NW = NC * NS
CH = 128
EPW = 5120
GCH = 64
NCH = EPW // GCH
NQ = 4
QS = 2560
A_PAD = QS + 8
HR = 640
BM = 1000
EPS = 1e-5


def _sc_prep(srcp, dstp):
    n_chunks = srcp.shape[0]
    cpw = n_chunks // NW
    mesh = plsc.VectorSubcoreMesh(core_axis_name="c", subcore_axis_name="s")

    @functools.partial(
        pl.kernel, mesh=mesh,
        out_type=(
            jax.ShapeDtypeStruct((NW, HR * 16), jnp.float32),
            jax.ShapeDtypeStruct((NW, NQ * EPW), jnp.int32),
            jax.ShapeDtypeStruct((NW, NQ * 2 * EPW), jnp.int32),
            jax.ShapeDtypeStruct((NW, NQ * 128), jnp.int32),
        ),
        scratch_types=[
            pltpu.VMEM((cpw, CH), jnp.int32),
            pltpu.VMEM((cpw, CH), jnp.int32),
            pltpu.VMEM((HR * 16,), jnp.float32),
            pltpu.VMEM((NQ * EPW,), jnp.int32),
            pltpu.VMEM((NQ * 2 * EPW,), jnp.int32),
            pltpu.VMEM((NQ * 128,), jnp.int32),
        ],
        compiler_params=pltpu.CompilerParams(needs_layout_passes=False),
    )
    def k(src_hbm, dst_hbm, hist_hbm, ps_hbm, pd_hbm, cnt_hbm,
          sbuf, dbuf, hist, bs, bd, cbuf):
        c = lax.axis_index("c")
        s = lax.axis_index("s")
        w = s * NC + c
        pltpu.sync_copy(src_hbm.at[pl.ds(w * cpw, cpw)], sbuf)
        pltpu.sync_copy(dst_hbm.at[pl.ds(w * cpw, cpw)], dbuf)

        zf = jnp.zeros((16,), jnp.float32)
        zi = jnp.zeros((16,), jnp.int32)
        trash = jnp.full((16,), 2 * QS, jnp.int32)

        def zero_body(r, carry):
            hist[pl.ds(r * 16, 16)] = zf
            return carry
        lax.fori_loop(0, HR, zero_body, 0)

        def fill_body(r, carry):
            bs[pl.ds(r * 16, 16)] = zi
            bd[pl.ds(r * 32, 16)] = trash
            bd[pl.ds(r * 32 + 16, 16)] = trash
            return carry
        lax.fori_loop(0, NQ * EPW // 16, fill_body, 0)

        ones = jnp.ones((16,), jnp.float32)

        def body(j, offs):
            offs = list(offs)
            for g in range(CH // 16):
                s16 = sbuf[j, pl.ds(g * 16, 16)]
                d16 = dbuf[j, pl.ds(g * 16, 16)]
                plsc.addupdate_scatter(hist, [d16], ones)
                for q in range(NQ):
                    if q == 0:
                        m = d16 < QS
                    elif q == NQ - 1:
                        m = d16 >= (NQ - 1) * QS
                    else:
                        m = jnp.logical_and(d16 >= q * QS,
                                            d16 < (q + 1) * QS)
                    cq = plsc.cumsum(m.astype(jnp.int32))
                    pos = cq + (q * EPW + offs[q] - 1)
                    plsc.store_scatter(bs, [pos], s16, mask=m)
                    dloc2 = 2 * (d16 - q * QS)
                    pos2 = 2 * cq + (2 * q * EPW + 2 * offs[q] - 2)
                    plsc.store_scatter(bd, [pos2], dloc2, mask=m)
                    plsc.store_scatter(bd, [pos2 + 1], dloc2 + 1, mask=m)
                    offs[q] = offs[q] + jnp.sum(m.astype(jnp.int32))
            return tuple(offs)

        offs = lax.fori_loop(0, cpw, body, (0, 0, 0, 0))

        pltpu.sync_copy(hist, hist_hbm.at[w])
        pltpu.sync_copy(bs, ps_hbm.at[w])
        pltpu.sync_copy(bd, pd_hbm.at[w])
        for q in range(NQ):
            for g in range(8):
                cbuf[pl.ds(q * 128 + g * 16, 16)] = jnp.full(
                    (16,), offs[q], jnp.int32)
        pltpu.sync_copy(cbuf, cnt_hbm.at[w])

    return k(srcp, dstp)


def _sc_aggregate(hp, psrc, pdst, cnt):
    mesh = plsc.VectorSubcoreMesh(core_axis_name="c", subcore_axis_name="s")

    @functools.partial(
        pl.kernel, mesh=mesh,
        out_type=jax.ShapeDtypeStruct((2 * N, 128), jnp.float32),
        scratch_types=[
            pltpu.VMEM((NCH // 2, GCH), jnp.int32),
            pltpu.VMEM((NCH // 2, 2 * GCH), jnp.int32),
            pltpu.VMEM((128,), jnp.int32),
            pltpu.VMEM((3, GCH, 2, 128), jnp.float32),
            pltpu.VMEM_SHARED((2 * A_PAD, 128), jnp.float32),
            pltpu.SemaphoreType.DMA((3,)),
            pltpu.SemaphoreType.DMA((3,)),
        ],
        compiler_params=pltpu.CompilerParams(needs_layout_passes=False),
    )
    def k(hp_hbm, hp2_hbm, ps_hbm, pd_hbm, cnt_hbm, out_hbm, sidx, didx,
          cbuf, rows, acc, sem, ssem):
        c = lax.axis_index("c")
        s = lax.axis_index("s")

        def init_drain(qrows2, base2, drain):
            rps = (qrows2 // NS) & ~7
            tail = qrows2 - NS * rps
            r0 = s * rps
            if drain:
                pltpu.sync_copy(acc.at[pl.ds(r0, rps)],
                                out_hbm.at[pl.ds(base2 + r0, rps)])
            else:
                pltpu.sync_copy(hp2_hbm.at[pl.ds(base2 + r0, rps)],
                                acc.at[pl.ds(r0, rps)])
            if tail:
                @pl.when(s == NS - 1)
                def _():
                    if drain:
                        pltpu.sync_copy(
                            acc.at[pl.ds(NS * rps, tail)],
                            out_hbm.at[pl.ds(base2 + NS * rps, tail)])
                    else:
                        pltpu.sync_copy(
                            hp2_hbm.at[pl.ds(base2 + NS * rps, tail)],
                            acc.at[pl.ds(NS * rps, tail)])

        def start_gather(j):
            b = lax.rem(j, 3)
            pltpu.async_copy(hp_hbm.at[sidx.at[j]], rows.at[b], sem.at[b])

        def wait_gather(j):
            b = lax.rem(j, 3)
            pltpu.make_async_copy(hp_hbm.at[sidx.at[j]], rows.at[b],
                                  sem.at[b]).wait()

        def start_scatter(j):
            b = lax.rem(j, 3)
            pltpu.async_copy(rows.at[b].reshape(2 * GCH, 128),
                             acc.at[didx.at[j]], ssem.at[b], add=True)

        def wait_scatter(j):
            b = lax.rem(j, 3)
            pltpu.make_async_copy(rows.at[b].reshape(2 * GCH, 128),
                                  acc.at[didx.at[j]], ssem.at[b]).wait()

        HCH = NCH // 2
        for q in range(2):
            for cc in range(NC):
                g = 2 * cc + q
                qrows = min(N - g * QS, QS)

                @pl.when(c == cc)
                def _():
                    init_drain(2 * qrows, 2 * g * QS, drain=False)
            plsc.subcore_barrier()

            for t in range(2):
                li = (2 * s + t) * NQ + 2 * c + q
                pltpu.sync_copy(cnt_hbm.at[li], cbuf)
                n = jnp.max(cbuf[pl.ds(0, 16)])
                trips = lax.shift_right_logical(n + (GCH - 1), 6)
                for p in range(2):
                    tp = jnp.clip(trips - p * HCH, 0, HCH)

                    @pl.when(tp > 0)
                    def _():
                        pltpu.sync_copy(ps_hbm.at[li * 2 + p], sidx)
                        pltpu.sync_copy(pd_hbm.at[li * 2 + p], didx)
                        start_gather(0)

                        def body(j, carry):
                            @pl.when(j > 0)
                            def _():
                                start_gather(j)
                            wait_gather(j)
                            start_scatter(j)
                            wait_scatter(j)
                            return carry
                        lax.fori_loop(0, tp, body, 0)

            plsc.subcore_barrier()
            for cc in range(NC):
                g = 2 * cc + q
                qrows = min(N - g * QS, QS)

                @pl.when(c == cc)
                def _():
                    init_drain(2 * qrows, 2 * g * QS, drain=True)
            if q == 0:
                plsc.subcore_barrier()

    return k(hp.reshape(N, 2, 128), hp.reshape(2 * N, 128), psrc, pdst,
             cnt).reshape(N, D)


def _tc_dinv(parts):
    def body(p_ref, o_ref):
        deg = jnp.sum(p_ref[...], axis=0, keepdims=True) + 1.0
        o_ref[...] = lax.rsqrt(deg)
    return pl.pallas_call(
        body,
        out_shape=jax.ShapeDtypeStruct((1, parts.shape[1]), jnp.float32),
    )(parts)


def _tc_pre(x, W, dinv):
    def body(x_ref, w_ref, dv_ref, o_ref):
        h = jnp.dot(x_ref[...], w_ref[...], preferred_element_type=jnp.float32)
        o_ref[...] = dv_ref[...] * h
    grid = N // BM
    return pl.pallas_call(
        body,
        grid=(grid,),
        in_specs=[
            pl.BlockSpec((BM, D), lambda i: (i, 0)),
            pl.BlockSpec((D, D), lambda i: (0, 0)),
            pl.BlockSpec((BM, 1), lambda i: (i, 0)),
        ],
        out_specs=pl.BlockSpec((BM, D), lambda i: (i, 0)),
        out_shape=jax.ShapeDtypeStruct((N, D), jnp.float32),
    )(x, W, dinv)


def _tc_stats(seg, dinv, b):
    def body(s_ref, dv_ref, b_ref, o_ref):
        t = dv_ref[...] * s_ref[...] + b_ref[...]
        @pl.when(pl.program_id(0) == 0)
        def _():
            o_ref[...] = jnp.zeros_like(o_ref)
        o_ref[0:1, :] += jnp.sum(t, axis=0, keepdims=True)
        o_ref[1:2, :] += jnp.sum(t * t, axis=0, keepdims=True)
    grid = N // BM
    return pl.pallas_call(
        body,
        grid=(grid,),
        in_specs=[
            pl.BlockSpec((BM, D), lambda i: (i, 0)),
            pl.BlockSpec((BM, 1), lambda i: (i, 0)),
            pl.BlockSpec((1, D), lambda i: (0, 0)),
        ],
        out_specs=pl.BlockSpec((8, D), lambda i: (0, 0)),
        out_shape=jax.ShapeDtypeStruct((8, D), jnp.float32),
    )(seg, dinv, b)


def _tc_bn_relu_mm(seg, dinv, b, stats, g, be, W):
    def body(s_ref, dv_ref, b_ref, st_ref, g_ref, be_ref, w_ref, o_ref):
        t = dv_ref[...] * s_ref[...] + b_ref[...]
        mu = st_ref[0:1, :] * (1.0 / N)
        var = st_ref[1:2, :] * (1.0 / N) - mu * mu
        u = g_ref[...] * (t - mu) * lax.rsqrt(var + EPS) + be_ref[...]
        u = jnp.maximum(u, 0.0)
        h = jnp.dot(u, w_ref[...], preferred_element_type=jnp.float32)
        o_ref[...] = dv_ref[...] * h
    grid = N // BM
    return pl.pallas_call(
        body,
        grid=(grid,),
        in_specs=[
            pl.BlockSpec((BM, D), lambda i: (i, 0)),
            pl.BlockSpec((BM, 1), lambda i: (i, 0)),
            pl.BlockSpec((1, D), lambda i: (0, 0)),
            pl.BlockSpec((8, D), lambda i: (0, 0)),
            pl.BlockSpec((1, D), lambda i: (0, 0)),
            pl.BlockSpec((1, D), lambda i: (0, 0)),
            pl.BlockSpec((D, D), lambda i: (0, 0)),
        ],
        out_specs=pl.BlockSpec((BM, D), lambda i: (i, 0)),
        out_shape=jax.ShapeDtypeStruct((N, D), jnp.float32),
    )(seg, dinv, b, stats, g, be, W)


def _tc_relu_mm(seg, dinv, b, W):
    def body(s_ref, dv_ref, b_ref, w_ref, o_ref):
        u = jnp.maximum(dv_ref[...] * s_ref[...] + b_ref[...], 0.0)
        h = jnp.dot(u, w_ref[...], preferred_element_type=jnp.float32)
        o_ref[...] = dv_ref[...] * h
    grid = N // BM
    return pl.pallas_call(
        body,
        grid=(grid,),
        in_specs=[
            pl.BlockSpec((BM, D), lambda i: (i, 0)),
            pl.BlockSpec((BM, 1), lambda i: (i, 0)),
            pl.BlockSpec((1, D), lambda i: (0, 0)),
            pl.BlockSpec((D, D), lambda i: (0, 0)),
        ],
        out_specs=pl.BlockSpec((BM, D), lambda i: (i, 0)),
        out_shape=jax.ShapeDtypeStruct((N, D), jnp.float32),
    )(seg, dinv, b, W)


def _tc_post(seg, dinv, b):
    def body(s_ref, dv_ref, b_ref, o_ref):
        o_ref[...] = dv_ref[...] * s_ref[...] + b_ref[...]
    grid = N // BM
    return pl.pallas_call(
        body,
        grid=(grid,),
        in_specs=[
            pl.BlockSpec((BM, D), lambda i: (i, 0)),
            pl.BlockSpec((BM, 1), lambda i: (i, 0)),
            pl.BlockSpec((1, D), lambda i: (0, 0)),
        ],
        out_specs=pl.BlockSpec((BM, D), lambda i: (i, 0)),
        out_shape=jax.ShapeDtypeStruct((N, D), jnp.float32),
    )(seg, dinv, b)


def kernel(x, edge_index, W0, b0, g0, be0, W1, b1, g1, be1, W2, b2):
    E = edge_index.shape[1]
    epad = -E % (NW * CH)
    src = edge_index[0]
    dst = edge_index[1]
    if epad:
        src = jnp.concatenate([src, jnp.zeros((epad,), jnp.int32)])
        dst = jnp.concatenate([dst, jnp.full((epad,), N, jnp.int32)])
    srcp = src.reshape(-1, CH)
    dstp = dst.reshape(-1, CH)

    parts, psrc, pdst, cnt = _sc_prep(srcp, dstp)
    psrc = psrc.reshape(NW * NQ * 2, NCH // 2, GCH)
    pdst = pdst.reshape(NW * NQ * 2, NCH // 2, 2 * GCH)
    cnt = cnt.reshape(NW * NQ, 128)
    dinv = _tc_dinv(parts).reshape(HR * 16, 1)[:N]

    b0r, g0r, be0r = b0.reshape(1, D), g0.reshape(1, D), be0.reshape(1, D)
    b1r, g1r, be1r = b1.reshape(1, D), g1.reshape(1, D), be1.reshape(1, D)
    b2r = b2.reshape(1, D)

    hp = _tc_pre(x, W0, dinv)
    s = _sc_aggregate(hp, psrc, pdst, cnt)
    st = _tc_stats(s, dinv, b0r)
    hp = _tc_bn_relu_mm(s, dinv, b0r, st, g0r, be0r, W1)

    s = _sc_aggregate(hp, psrc, pdst, cnt)
    st = _tc_stats(s, dinv, b1r)
    hp = _tc_bn_relu_mm(s, dinv, b1r, st, g1r, be1r, W2)

    s = _sc_aggregate(hp, psrc, pdst, cnt)
    hp = _tc_relu_mm(s, dinv, b2r, W2)

    s = _sc_aggregate(hp, psrc, pdst, cnt)
    return _tc_post(s, dinv, b2r)


# speedup vs baseline: 1.2072x; 1.2072x over previous
import functools

import jax
import jax.numpy as jnp
from jax import lax
from jax.experimental import pallas as pl
from jax.experimental.pallas import tpu as pltpu
from jax.experimental.pallas import tpu_sc as plsc

N = 10000
D = 256
NC = 2
NS = 16
NW = NC * NS
CH = 128
EPW = 5120
GCH = 64
NCH = EPW // GCH
NQ = 4
QS = 2560
A_PAD = QS + 8
HR = 640
BM = 1000
EPS = 1e-5


def _sc_prep(srcp, dstp):
    n_chunks = srcp.shape[0]
    cpw = n_chunks // NW
    mesh = plsc.VectorSubcoreMesh(core_axis_name="c", subcore_axis_name="s")

    @functools.partial(
        pl.kernel, mesh=mesh,
        out_type=(
            jax.ShapeDtypeStruct((NW, HR * 16), jnp.float32),
            jax.ShapeDtypeStruct((NW, NQ * EPW), jnp.int32),
            jax.ShapeDtypeStruct((NW, NQ * 2 * EPW), jnp.int32),
            jax.ShapeDtypeStruct((NW, NQ * 128), jnp.int32),
        ),
        scratch_types=[
            pltpu.VMEM((cpw, CH), jnp.int32),
            pltpu.VMEM((cpw, CH), jnp.int32),
            pltpu.VMEM((HR * 16,), jnp.float32),
            pltpu.VMEM((NQ * EPW,), jnp.int32),
            pltpu.VMEM((NQ * 2 * EPW,), jnp.int32),
            pltpu.VMEM((NQ * 128,), jnp.int32),
        ],
        compiler_params=pltpu.CompilerParams(needs_layout_passes=False),
    )
    def k(src_hbm, dst_hbm, hist_hbm, ps_hbm, pd_hbm, cnt_hbm,
          sbuf, dbuf, hist, bs, bd, cbuf):
        c = lax.axis_index("c")
        s = lax.axis_index("s")
        w = s * NC + c
        pltpu.sync_copy(src_hbm.at[pl.ds(w * cpw, cpw)], sbuf)
        pltpu.sync_copy(dst_hbm.at[pl.ds(w * cpw, cpw)], dbuf)

        zf = jnp.zeros((16,), jnp.float32)
        zi = jnp.zeros((16,), jnp.int32)
        trash = jnp.full((16,), 2 * QS, jnp.int32)

        def zero_body(r, carry):
            hist[pl.ds(r * 16, 16)] = zf
            return carry
        lax.fori_loop(0, HR, zero_body, 0)

        def fill_body(r, carry):
            bs[pl.ds(r * 16, 16)] = zi
            bd[pl.ds(r * 32, 16)] = trash
            bd[pl.ds(r * 32 + 16, 16)] = trash
            return carry
        lax.fori_loop(0, NQ * EPW // 16, fill_body, 0)

        ones = jnp.ones((16,), jnp.float32)

        def body(j, offs):
            offs = list(offs)
            for g in range(CH // 16):
                s16 = sbuf[j, pl.ds(g * 16, 16)]
                d16 = dbuf[j, pl.ds(g * 16, 16)]
                plsc.addupdate_scatter(hist, [d16], ones)
                for q in range(NQ):
                    if q == 0:
                        m = d16 < QS
                    elif q == NQ - 1:
                        m = d16 >= (NQ - 1) * QS
                    else:
                        m = jnp.logical_and(d16 >= q * QS,
                                            d16 < (q + 1) * QS)
                    cq = plsc.cumsum(m.astype(jnp.int32))
                    pos = cq + (q * EPW + offs[q] - 1)
                    plsc.store_scatter(bs, [pos], s16, mask=m)
                    dloc2 = 2 * (d16 - q * QS)
                    pos2 = 2 * cq + (2 * q * EPW + 2 * offs[q] - 2)
                    plsc.store_scatter(bd, [pos2], dloc2, mask=m)
                    plsc.store_scatter(bd, [pos2 + 1], dloc2 + 1, mask=m)
                    offs[q] = offs[q] + jnp.sum(m.astype(jnp.int32))
            return tuple(offs)

        offs = lax.fori_loop(0, cpw, body, (0, 0, 0, 0))

        pltpu.sync_copy(hist, hist_hbm.at[w])
        pltpu.sync_copy(bs, ps_hbm.at[w])
        pltpu.sync_copy(bd, pd_hbm.at[w])
        for q in range(NQ):
            for g in range(8):
                cbuf[pl.ds(q * 128 + g * 16, 16)] = jnp.full(
                    (16,), offs[q], jnp.int32)
        pltpu.sync_copy(cbuf, cnt_hbm.at[w])

    return k(srcp, dstp)


def _sc_aggregate(hp, psrc, pdst, cnt):
    mesh = plsc.VectorSubcoreMesh(core_axis_name="c", subcore_axis_name="s")

    @functools.partial(
        pl.kernel, mesh=mesh,
        out_type=jax.ShapeDtypeStruct((2 * N, 128), jnp.float32),
        scratch_types=[
            pltpu.VMEM((NCH // 2, GCH), jnp.int32),
            pltpu.VMEM((NCH // 2, 2 * GCH), jnp.int32),
            pltpu.VMEM((128,), jnp.int32),
            pltpu.VMEM((3, 2 * GCH, 128), jnp.float32),
            pltpu.VMEM_SHARED((2 * A_PAD, 128), jnp.float32),
            pltpu.SemaphoreType.DMA((3,)),
            pltpu.SemaphoreType.DMA((3,)),
        ],
        compiler_params=pltpu.CompilerParams(needs_layout_passes=False),
    )
    def k(hp_hbm, hp2_hbm, ps_hbm, pd_hbm, cnt_hbm, out_hbm, sidx, didx,
          cbuf, rows, acc, sem, ssem):
        c = lax.axis_index("c")
        s = lax.axis_index("s")

        def init_drain(qrows2, base2, drain):
            rps = (qrows2 // NS) & ~7
            tail = qrows2 - NS * rps
            r0 = s * rps
            if drain:
                pltpu.sync_copy(acc.at[pl.ds(r0, rps)],
                                out_hbm.at[pl.ds(base2 + r0, rps)])
            else:
                pltpu.sync_copy(hp2_hbm.at[pl.ds(base2 + r0, rps)],
                                acc.at[pl.ds(r0, rps)])
            if tail:
                @pl.when(s == NS - 1)
                def _():
                    if drain:
                        pltpu.sync_copy(
                            acc.at[pl.ds(NS * rps, tail)],
                            out_hbm.at[pl.ds(base2 + NS * rps, tail)])
                    else:
                        pltpu.sync_copy(
                            hp2_hbm.at[pl.ds(base2 + NS * rps, tail)],
                            acc.at[pl.ds(NS * rps, tail)])

        def start_gather(j):
            b = lax.rem(j, 3)
            pltpu.async_copy(hp_hbm.at[sidx.at[j]],
                             rows.at[b].reshape(GCH, 2, 128), sem.at[b])

        def wait_gather(j):
            b = lax.rem(j, 3)
            pltpu.make_async_copy(hp_hbm.at[sidx.at[j]],
                                  rows.at[b].reshape(GCH, 2, 128),
                                  sem.at[b]).wait()

        def start_scatter(j):
            b = lax.rem(j, 3)
            pltpu.async_copy(rows.at[b], acc.at[didx.at[j]], ssem.at[b],
                             add=True)

        def wait_scatter(j):
            b = lax.rem(j, 3)
            pltpu.make_async_copy(rows.at[b], acc.at[didx.at[j]],
                                  ssem.at[b]).wait()

        HCH = NCH // 2
        for q in range(2):
            for cc in range(NC):
                g = 2 * cc + q
                qrows = min(N - g * QS, QS)

                @pl.when(c == cc)
                def _():
                    init_drain(2 * qrows, 2 * g * QS, drain=False)
            plsc.subcore_barrier()

            for t in range(2):
                li = (2 * s + t) * NQ + 2 * c + q
                pltpu.sync_copy(cnt_hbm.at[li], cbuf)
                n = jnp.max(cbuf[pl.ds(0, 16)])
                trips = lax.shift_right_logical(n + (GCH - 1), 6)
                for p in range(2):
                    tp = jnp.clip(trips - p * HCH, 0, HCH)

                    @pl.when(tp > 0)
                    def _():
                        pltpu.sync_copy(ps_hbm.at[li * 2 + p], sidx)
                        pltpu.sync_copy(pd_hbm.at[li * 2 + p], didx)
                        start_gather(0)

                        def body(j, carry):
                            @pl.when(j + 1 < tp)
                            def _():
                                @pl.when(j >= 2)
                                def _():
                                    wait_scatter(j - 2)
                                start_gather(j + 1)
                            wait_gather(j)
                            start_scatter(j)
                            return carry
                        lax.fori_loop(0, tp, body, 0)
                        wait_scatter(tp - 1)

                        @pl.when(tp >= 2)
                        def _():
                            wait_scatter(tp - 2)

                        @pl.when(tp >= 3)
                        def _():
                            wait_scatter(tp - 3)

            plsc.subcore_barrier()
            for cc in range(NC):
                g = 2 * cc + q
                qrows = min(N - g * QS, QS)

                @pl.when(c == cc)
                def _():
                    init_drain(2 * qrows, 2 * g * QS, drain=True)
            if q == 0:
                plsc.subcore_barrier()

    return k(hp.reshape(N, 2, 128), hp.reshape(2 * N, 128), psrc, pdst,
             cnt).reshape(N, D)


def _tc_dinv(parts):
    def body(p_ref, o_ref):
        deg = jnp.sum(p_ref[...], axis=0, keepdims=True) + 1.0
        o_ref[...] = lax.rsqrt(deg)
    return pl.pallas_call(
        body,
        out_shape=jax.ShapeDtypeStruct((1, parts.shape[1]), jnp.float32),
    )(parts)


def _tc_pre(x, W, dinv):
    def body(x_ref, w_ref, dv_ref, o_ref):
        h = jnp.dot(x_ref[...], w_ref[...], preferred_element_type=jnp.float32)
        o_ref[...] = dv_ref[...] * h
    grid = N // BM
    return pl.pallas_call(
        body,
        grid=(grid,),
        in_specs=[
            pl.BlockSpec((BM, D), lambda i: (i, 0)),
            pl.BlockSpec((D, D), lambda i: (0, 0)),
            pl.BlockSpec((BM, 1), lambda i: (i, 0)),
        ],
        out_specs=pl.BlockSpec((BM, D), lambda i: (i, 0)),
        out_shape=jax.ShapeDtypeStruct((N, D), jnp.float32),
    )(x, W, dinv)


def _tc_stats(seg, dinv, b):
    def body(s_ref, dv_ref, b_ref, o_ref):
        t = dv_ref[...] * s_ref[...] + b_ref[...]
        @pl.when(pl.program_id(0) == 0)
        def _():
            o_ref[...] = jnp.zeros_like(o_ref)
        o_ref[0:1, :] += jnp.sum(t, axis=0, keepdims=True)
        o_ref[1:2, :] += jnp.sum(t * t, axis=0, keepdims=True)
    grid = N // BM
    return pl.pallas_call(
        body,
        grid=(grid,),
        in_specs=[
            pl.BlockSpec((BM, D), lambda i: (i, 0)),
            pl.BlockSpec((BM, 1), lambda i: (i, 0)),
            pl.BlockSpec((1, D), lambda i: (0, 0)),
        ],
        out_specs=pl.BlockSpec((8, D), lambda i: (0, 0)),
        out_shape=jax.ShapeDtypeStruct((8, D), jnp.float32),
    )(seg, dinv, b)


def _tc_bn_relu_mm(seg, dinv, b, stats, g, be, W):
    def body(s_ref, dv_ref, b_ref, st_ref, g_ref, be_ref, w_ref, o_ref):
        t = dv_ref[...] * s_ref[...] + b_ref[...]
        mu = st_ref[0:1, :] * (1.0 / N)
        var = st_ref[1:2, :] * (1.0 / N) - mu * mu
        u = g_ref[...] * (t - mu) * lax.rsqrt(var + EPS) + be_ref[...]
        u = jnp.maximum(u, 0.0)
        h = jnp.dot(u, w_ref[...], preferred_element_type=jnp.float32)
        o_ref[...] = dv_ref[...] * h
    grid = N // BM
    return pl.pallas_call(
        body,
        grid=(grid,),
        in_specs=[
            pl.BlockSpec((BM, D), lambda i: (i, 0)),
            pl.BlockSpec((BM, 1), lambda i: (i, 0)),
            pl.BlockSpec((1, D), lambda i: (0, 0)),
            pl.BlockSpec((8, D), lambda i: (0, 0)),
            pl.BlockSpec((1, D), lambda i: (0, 0)),
            pl.BlockSpec((1, D), lambda i: (0, 0)),
            pl.BlockSpec((D, D), lambda i: (0, 0)),
        ],
        out_specs=pl.BlockSpec((BM, D), lambda i: (i, 0)),
        out_shape=jax.ShapeDtypeStruct((N, D), jnp.float32),
    )(seg, dinv, b, stats, g, be, W)


def _tc_relu_mm(seg, dinv, b, W):
    def body(s_ref, dv_ref, b_ref, w_ref, o_ref):
        u = jnp.maximum(dv_ref[...] * s_ref[...] + b_ref[...], 0.0)
        h = jnp.dot(u, w_ref[...], preferred_element_type=jnp.float32)
        o_ref[...] = dv_ref[...] * h
    grid = N // BM
    return pl.pallas_call(
        body,
        grid=(grid,),
        in_specs=[
            pl.BlockSpec((BM, D), lambda i: (i, 0)),
            pl.BlockSpec((BM, 1), lambda i: (i, 0)),
            pl.BlockSpec((1, D), lambda i: (0, 0)),
            pl.BlockSpec((D, D), lambda i: (0, 0)),
        ],
        out_specs=pl.BlockSpec((BM, D), lambda i: (i, 0)),
        out_shape=jax.ShapeDtypeStruct((N, D), jnp.float32),
    )(seg, dinv, b, W)


def _tc_post(seg, dinv, b):
    def body(s_ref, dv_ref, b_ref, o_ref):
        o_ref[...] = dv_ref[...] * s_ref[...] + b_ref[...]
    grid = N // BM
    return pl.pallas_call(
        body,
        grid=(grid,),
        in_specs=[
            pl.BlockSpec((BM, D), lambda i: (i, 0)),
            pl.BlockSpec((BM, 1), lambda i: (i, 0)),
            pl.BlockSpec((1, D), lambda i: (0, 0)),
        ],
        out_specs=pl.BlockSpec((BM, D), lambda i: (i, 0)),
        out_shape=jax.ShapeDtypeStruct((N, D), jnp.float32),
    )(seg, dinv, b)


def kernel(x, edge_index, W0, b0, g0, be0, W1, b1, g1, be1, W2, b2):
    E = edge_index.shape[1]
    epad = -E % (NW * CH)
    src = edge_index[0]
    dst = edge_index[1]
    if epad:
        src = jnp.concatenate([src, jnp.zeros((epad,), jnp.int32)])
        dst = jnp.concatenate([dst, jnp.full((epad,), N, jnp.int32)])
    srcp = src.reshape(-1, CH)
    dstp = dst.reshape(-1, CH)

    parts, psrc, pdst, cnt = _sc_prep(srcp, dstp)
    psrc = psrc.reshape(NW * NQ * 2, NCH // 2, GCH)
    pdst = pdst.reshape(NW * NQ * 2, NCH // 2, 2 * GCH)
    cnt = cnt.reshape(NW * NQ, 128)
    dinv = _tc_dinv(parts).reshape(HR * 16, 1)[:N]

    b0r, g0r, be0r = b0.reshape(1, D), g0.reshape(1, D), be0.reshape(1, D)
    b1r, g1r, be1r = b1.reshape(1, D), g1.reshape(1, D), be1.reshape(1, D)
    b2r = b2.reshape(1, D)

    hp = _tc_pre(x, W0, dinv)
    s = _sc_aggregate(hp, psrc, pdst, cnt)
    st = _tc_stats(s, dinv, b0r)
    hp = _tc_bn_relu_mm(s, dinv, b0r, st, g0r, be0r, W1)

    s = _sc_aggregate(hp, psrc, pdst, cnt)
    st = _tc_stats(s, dinv, b1r)
    hp = _tc_bn_relu_mm(s, dinv, b1r, st, g1r, be1r, W2)

    s = _sc_aggregate(hp, psrc, pdst, cnt)
    hp = _tc_relu_mm(s, dinv, b2r, W2)

    s = _sc_aggregate(hp, psrc, pdst, cnt)
    return _tc_post(s, dinv, b2r)


# speedup vs baseline: 1.8965x; 1.5710x over previous
import functools

import jax
import jax.numpy as jnp
from jax import lax
from jax.experimental import pallas as pl
from jax.experimental.pallas import tpu as pltpu
from jax.experimental.pallas import tpu_sc as plsc

N = 10000
D = 256
H = 128
NC = 2
NS = 16
NW = NC * NS
CH = 128
N_PAD = N + 16
RPS = 624
HR = 640
BM = 1000
EPS = 1e-5


def _sc_degree(dstp):
    n_chunks = dstp.shape[0]
    cpw = n_chunks // NW
    mesh = plsc.VectorSubcoreMesh(core_axis_name="c", subcore_axis_name="s")

    @functools.partial(
        pl.kernel, mesh=mesh,
        out_type=jax.ShapeDtypeStruct((NW, HR * 16), jnp.float32),
        scratch_types=[
            pltpu.VMEM((CH,), jnp.int32),
            pltpu.VMEM((HR * 16,), jnp.float32),
        ],
        compiler_params=pltpu.CompilerParams(needs_layout_passes=False),
    )
    def k(dst_hbm, out_hbm, idxv, hist):
        c = lax.axis_index("c")
        s = lax.axis_index("s")
        w = s * NC + c
        z = jnp.zeros((16,), jnp.float32)

        def zero_body(r, carry):
            hist[pl.ds(r * 16, 16)] = z
            return carry
        lax.fori_loop(0, HR, zero_body, 0)

        ones = jnp.ones((16,), jnp.float32)

        def body(j, carry):
            pltpu.sync_copy(dst_hbm.at[w * cpw + j], idxv)
            for kk in range(CH // 16):
                ii = idxv[pl.ds(kk * 16, 16)]
                plsc.addupdate_scatter(hist, [ii], ones)
            return carry
        lax.fori_loop(0, cpw, body, 0)
        pltpu.sync_copy(hist, out_hbm.at[w])

    return k(dstp)


def _sc_aggregate(hp, srcp, dstp):
    n_chunks = srcp.shape[0]
    cps = n_chunks // NS
    mesh = plsc.VectorSubcoreMesh(core_axis_name="c", subcore_axis_name="s")

    @functools.partial(
        pl.kernel, mesh=mesh,
        out_type=jax.ShapeDtypeStruct((NC, N, H), jnp.float32),
        scratch_types=[
            pltpu.VMEM((cps // 2, CH), jnp.int32),
            pltpu.VMEM((cps // 2, CH), jnp.int32),
            pltpu.VMEM((2, CH, H), jnp.float32),
            pltpu.VMEM_SHARED((N_PAD, H), jnp.float32),
            pltpu.SemaphoreType.DMA((2,)),
            pltpu.SemaphoreType.DMA((2,)),
        ],
        compiler_params=pltpu.CompilerParams(needs_layout_passes=False),
    )
    def k(hp_hbm, src_hbm, dst_hbm, out_hbm, sidx, didx, rows, acc, sem,
          ssem):
        c = lax.axis_index("c")
        s = lax.axis_index("s")
        r0 = s * RPS
        pltpu.sync_copy(hp_hbm.at[c, pl.ds(r0, RPS)], acc.at[pl.ds(r0, RPS)])

        @pl.when(s == NS - 1)
        def _():
            pltpu.sync_copy(hp_hbm.at[c, pl.ds(NS * RPS, N - NS * RPS)],
                            acc.at[pl.ds(NS * RPS, N - NS * RPS)])
        plsc.subcore_barrier()

        def start_gather(j):
            b = lax.rem(j, 2)
            pltpu.async_copy(hp_hbm.at[c].at[sidx.at[j]], rows.at[b],
                             sem.at[b])

        def wait_gather(j):
            b = lax.rem(j, 2)
            pltpu.make_async_copy(hp_hbm.at[c].at[sidx.at[j]], rows.at[b],
                                  sem.at[b]).wait()

        def start_scatter(j):
            b = lax.rem(j, 2)
            pltpu.async_copy(rows.at[b], acc.at[didx.at[j]], ssem.at[b],
                             add=True)

        def wait_scatter(j):
            b = lax.rem(j, 2)
            pltpu.make_async_copy(rows.at[b], acc.at[didx.at[j]],
                                  ssem.at[b]).wait()

        hc = cps // 2
        for p in range(2):
            pltpu.sync_copy(src_hbm.at[pl.ds(s * cps + p * hc, hc)], sidx)
            pltpu.sync_copy(dst_hbm.at[pl.ds(s * cps + p * hc, hc)], didx)
            start_gather(0)

            def body(j, carry):
                @pl.when(j + 1 < hc)
                def _():
                    start_gather(j + 1)
                wait_gather(j)
                pltpu.sync_copy(rows.at[lax.rem(j, 2)], acc.at[didx.at[j]],
                                add=True)
                return carry
            lax.fori_loop(0, hc, body, 0)

        plsc.subcore_barrier()
        pltpu.sync_copy(acc.at[pl.ds(r0, RPS)], out_hbm.at[c, pl.ds(r0, RPS)])

        @pl.when(s == NS - 1)
        def _():
            pltpu.sync_copy(acc.at[pl.ds(NS * RPS, N - NS * RPS)],
                            out_hbm.at[c, pl.ds(NS * RPS, N - NS * RPS)])

    return k(hp, srcp, dstp)


def _tc_dinv(parts):
    def body(p_ref, o_ref):
        deg = jnp.sum(p_ref[...], axis=0, keepdims=True) + 1.0
        o_ref[...] = lax.rsqrt(deg)
    return pl.pallas_call(
        body,
        out_shape=jax.ShapeDtypeStruct((1, parts.shape[1]), jnp.float32),
    )(parts)


def _split_store(o_ref, hp):
    o_ref[0] = hp[:, :H]
    o_ref[1] = hp[:, H:]


def _tc_pre(x, W, dinv):
    def body(x_ref, w_ref, dv_ref, o_ref):
        h = jnp.dot(x_ref[...], w_ref[...], preferred_element_type=jnp.float32)
        _split_store(o_ref, dv_ref[...] * h)
    grid = N // BM
    return pl.pallas_call(
        body,
        grid=(grid,),
        in_specs=[
            pl.BlockSpec((BM, D), lambda i: (i, 0)),
            pl.BlockSpec((D, D), lambda i: (0, 0)),
            pl.BlockSpec((BM, 1), lambda i: (i, 0)),
        ],
        out_specs=pl.BlockSpec((NC, BM, H), lambda i: (0, i, 0)),
        out_shape=jax.ShapeDtypeStruct((NC, N, H), jnp.float32),
    )(x, W, dinv)


def _combine(s_ref, dv_ref, b_ref):
    seg = jnp.concatenate([s_ref[0], s_ref[1]], axis=1)
    return dv_ref[...] * seg + b_ref[...]


def _tc_stats(segp, dinv, b):
    def body(s_ref, dv_ref, b_ref, o_ref):
        t = _combine(s_ref, dv_ref, b_ref)
        @pl.when(pl.program_id(0) == 0)
        def _():
            o_ref[...] = jnp.zeros_like(o_ref)
        o_ref[0:1, :] += jnp.sum(t, axis=0, keepdims=True)
        o_ref[1:2, :] += jnp.sum(t * t, axis=0, keepdims=True)
    grid = N // BM
    return pl.pallas_call(
        body,
        grid=(grid,),
        in_specs=[
            pl.BlockSpec((NC, BM, H), lambda i: (0, i, 0)),
            pl.BlockSpec((BM, 1), lambda i: (i, 0)),
            pl.BlockSpec((1, D), lambda i: (0, 0)),
        ],
        out_specs=pl.BlockSpec((8, D), lambda i: (0, 0)),
        out_shape=jax.ShapeDtypeStruct((8, D), jnp.float32),
    )(segp, dinv, b)


def _tc_bn_relu_mm(segp, dinv, b, stats, g, be, W):
    def body(s_ref, dv_ref, b_ref, st_ref, g_ref, be_ref, w_ref, o_ref):
        t = _combine(s_ref, dv_ref, b_ref)
        mu = st_ref[0:1, :] * (1.0 / N)
        var = st_ref[1:2, :] * (1.0 / N) - mu * mu
        u = g_ref[...] * (t - mu) * lax.rsqrt(var + EPS) + be_ref[...]
        u = jnp.maximum(u, 0.0)
        h = jnp.dot(u, w_ref[...], preferred_element_type=jnp.float32)
        _split_store(o_ref, dv_ref[...] * h)
    grid = N // BM
    return pl.pallas_call(
        body,
        grid=(grid,),
        in_specs=[
            pl.BlockSpec((NC, BM, H), lambda i: (0, i, 0)),
            pl.BlockSpec((BM, 1), lambda i: (i, 0)),
            pl.BlockSpec((1, D), lambda i: (0, 0)),
            pl.BlockSpec((8, D), lambda i: (0, 0)),
            pl.BlockSpec((1, D), lambda i: (0, 0)),
            pl.BlockSpec((1, D), lambda i: (0, 0)),
            pl.BlockSpec((D, D), lambda i: (0, 0)),
        ],
        out_specs=pl.BlockSpec((NC, BM, H), lambda i: (0, i, 0)),
        out_shape=jax.ShapeDtypeStruct((NC, N, H), jnp.float32),
    )(segp, dinv, b, stats, g, be, W)


def _tc_relu_mm(segp, dinv, b, W):
    def body(s_ref, dv_ref, b_ref, w_ref, o_ref):
        u = jnp.maximum(_combine(s_ref, dv_ref, b_ref), 0.0)
        h = jnp.dot(u, w_ref[...], preferred_element_type=jnp.float32)
        _split_store(o_ref, dv_ref[...] * h)
    grid = N // BM
    return pl.pallas_call(
        body,
        grid=(grid,),
        in_specs=[
            pl.BlockSpec((NC, BM, H), lambda i: (0, i, 0)),
            pl.BlockSpec((BM, 1), lambda i: (i, 0)),
            pl.BlockSpec((1, D), lambda i: (0, 0)),
            pl.BlockSpec((D, D), lambda i: (0, 0)),
        ],
        out_specs=pl.BlockSpec((NC, BM, H), lambda i: (0, i, 0)),
        out_shape=jax.ShapeDtypeStruct((NC, N, H), jnp.float32),
    )(segp, dinv, b, W)


def _tc_post(segp, dinv, b):
    def body(s_ref, dv_ref, b_ref, o_ref):
        o_ref[...] = _combine(s_ref, dv_ref, b_ref)
    grid = N // BM
    return pl.pallas_call(
        body,
        grid=(grid,),
        in_specs=[
            pl.BlockSpec((NC, BM, H), lambda i: (0, i, 0)),
            pl.BlockSpec((BM, 1), lambda i: (i, 0)),
            pl.BlockSpec((1, D), lambda i: (0, 0)),
        ],
        out_specs=pl.BlockSpec((BM, D), lambda i: (i, 0)),
        out_shape=jax.ShapeDtypeStruct((N, D), jnp.float32),
    )(segp, dinv, b)


def kernel(x, edge_index, W0, b0, g0, be0, W1, b1, g1, be1, W2, b2):
    E = edge_index.shape[1]
    epad = -E % (NW * CH)
    src = edge_index[0]
    dst = edge_index[1]
    if epad:
        src = jnp.concatenate([src, jnp.zeros((epad,), jnp.int32)])
        dst = jnp.concatenate([dst, jnp.full((epad,), N, jnp.int32)])
    srcp = src.reshape(-1, CH)
    dstp = dst.reshape(-1, CH)

    parts = _sc_degree(dstp)
    dinv = _tc_dinv(parts).reshape(HR * 16, 1)[:N]

    b0r, g0r, be0r = b0.reshape(1, D), g0.reshape(1, D), be0.reshape(1, D)
    b1r, g1r, be1r = b1.reshape(1, D), g1.reshape(1, D), be1.reshape(1, D)
    b2r = b2.reshape(1, D)

    hp = _tc_pre(x, W0, dinv)
    s = _sc_aggregate(hp, srcp, dstp)
    st = _tc_stats(s, dinv, b0r)
    hp = _tc_bn_relu_mm(s, dinv, b0r, st, g0r, be0r, W1)

    s = _sc_aggregate(hp, srcp, dstp)
    st = _tc_stats(s, dinv, b1r)
    hp = _tc_bn_relu_mm(s, dinv, b1r, st, g1r, be1r, W2)

    s = _sc_aggregate(hp, srcp, dstp)
    hp = _tc_relu_mm(s, dinv, b2r, W2)

    s = _sc_aggregate(hp, srcp, dstp)
    return _tc_post(s, dinv, b2r)
